# baseline (speedup 1.0000x reference)
import functools

import jax
import jax.numpy as jnp
import numpy as np
from jax import lax
from jax.experimental import pallas as pl
from jax.experimental.pallas import tpu as pltpu
from jax.experimental.pallas import tpu_sc as plsc

N = 10000
E = 320000
DIM = 128
H = 8
D = DIM // H
BLK = 2000
EPS = 1e-5

NW = 32
ROWS_PER_TILE = E // NW
CHUNK = 80
STEPS = ROWS_PER_TILE // CHUNK


def _sc_gather_body(a_hbm, src_hbm, dst_hbm, srcf_hbm, dstf_hbm,
                    idxbuf, rb0, rb1, sem0, sem1):
    wid = lax.axis_index("s") * 2 + lax.axis_index("c")
    for idx_hbm, out_hbm in ((src_hbm, srcf_hbm), (dst_hbm, dstf_hbm)):
        pltpu.sync_copy(idx_hbm.at[wid], idxbuf)
        pltpu.async_copy(a_hbm.at[idxbuf.at[0]], rb0, sem0)

        def body(j, _):
            even = (j % 2) == 0
            more = j + 1 < STEPS
            out_at = out_hbm.at[pl.ds(wid * ROWS_PER_TILE + j * CHUNK, CHUNK)]

            @pl.when(even)
            def _():
                pltpu.make_async_copy(a_hbm.at[idxbuf.at[j]], rb0, sem0).wait()

                @pl.when(more)
                def _():
                    pltpu.async_copy(a_hbm.at[idxbuf.at[j + 1]], rb1, sem1)

                pltpu.sync_copy(rb0, out_at)

            @pl.when(jnp.logical_not(even))
            def _():
                pltpu.make_async_copy(a_hbm.at[idxbuf.at[j]], rb1, sem1).wait()

                @pl.when(more)
                def _():
                    pltpu.async_copy(a_hbm.at[idxbuf.at[j + 1]], rb0, sem0)

                pltpu.sync_copy(rb1, out_at)

            return 0

        lax.fori_loop(0, STEPS, body, 0)


def _sc_gather(atom, src2d, dst2d):
    mesh = plsc.VectorSubcoreMesh(core_axis_name="c", subcore_axis_name="s")
    fn = functools.partial(
        pl.kernel, mesh=mesh,
        out_type=[
            jax.ShapeDtypeStruct((E, DIM), jnp.float32),
            jax.ShapeDtypeStruct((E, DIM), jnp.float32),
        ],
        scratch_types=[
            pltpu.VMEM((STEPS, CHUNK), jnp.int32),
            pltpu.VMEM((CHUNK, DIM), jnp.float32),
            pltpu.VMEM((CHUNK, DIM), jnp.float32),
            pltpu.SemaphoreType.DMA,
            pltpu.SemaphoreType.DMA,
        ],
    )(_sc_gather_body)
    return fn(atom, src2d, dst2d)


def _block_diag8(x):
    out = jnp.zeros((DIM, DIM), jnp.float32)
    for h in range(H):
        out = out.at[h * D:(h + 1) * D, h * D:(h + 1) * D].set(x)
    return out


def _fold_weights(params):
    Wq, bq = params["Wq"], params["bq"]
    Wk, bk = params["Wk"], params["bk"]
    Wv, bv = params["Wv"], params["bv"]
    We, be = params["We"], params["be"]

    def fold(W1, b1, Wx, bx):
        U1a = _block_diag8(W1[:, 0:D].T)
        U1b = _block_diag8(W1[:, D:2 * D].T)
        U1c = _block_diag8(W1[:, 2 * D:3 * D].T)
        Msrc = Wx.T @ U1a
        Mdst = Wx.T @ U1b
        Medge = We.T @ U1c
        bias = bx @ U1a + bx @ U1b + be @ U1c + jnp.tile(b1, H)
        return Msrc, Mdst, Medge, bias

    M1, M2, M3, b_hk = fold(params["ku_W1"], params["ku_b1"], Wk, bk)
    M4, M5, M6, b_hv = fold(params["mu_W1"], params["mu_b1"], Wv, bv)
    BD2k = _block_diag8(params["ku_W2"].T)
    BD2v = _block_diag8(params["mu_W2"].T)
    b_kt = jnp.tile(params["ku_b2"], H)
    b_vt = jnp.tile(params["mu_b2"], H)

    Wmats = jnp.stack([Wq.T, M1, M2, M3, M4, M5, M6, BD2k, BD2v])
    Bvecs = jnp.stack([bq, b_hk, b_hv, b_kt, b_vt,
                       jnp.zeros_like(bq), jnp.zeros_like(bq),
                       jnp.zeros_like(bq)])
    return Wmats, Bvecs


NPAD = 10240
SUB = NPAD // 16


def _sc_scatter_body(m_hbm, dst_hbm, z128_hbm,
                     acc_hbm,
                     idxbuf, rb0, rb1, shared, sem0, sem1):
    cid = lax.axis_index("c")
    sid = lax.axis_index("s")
    wid = sid * 2 + cid
    nchunk = SUB // CHUNK

    pltpu.sync_copy(z128_hbm, rb0)
    for t in range(nchunk):
        pltpu.sync_copy(rb0, shared.at[pl.ds(sid * SUB + t * CHUNK, CHUNK)])
    pltpu.sync_copy(dst_hbm.at[wid], idxbuf)
    plsc.subcore_barrier()

    base = wid * ROWS_PER_TILE
    pltpu.async_copy(m_hbm.at[pl.ds(base, CHUNK)], rb0, sem0)

    def body(j, _):
        even = (j % 2) == 0
        more = j + 1 < STEPS

        @pl.when(even)
        def _():
            pltpu.make_async_copy(
                m_hbm.at[pl.ds(base + j * CHUNK, CHUNK)], rb0, sem0).wait()

            @pl.when(more)
            def _():
                pltpu.async_copy(
                    m_hbm.at[pl.ds(base + (j + 1) * CHUNK, CHUNK)], rb1, sem1)

            pltpu.sync_copy(rb0, shared.at[idxbuf.at[j]], add=True)

        @pl.when(jnp.logical_not(even))
        def _():
            pltpu.make_async_copy(
                m_hbm.at[pl.ds(base + j * CHUNK, CHUNK)], rb1, sem1).wait()

            @pl.when(more)
            def _():
                pltpu.async_copy(
                    m_hbm.at[pl.ds(base + (j + 1) * CHUNK, CHUNK)], rb0, sem0)

            pltpu.sync_copy(rb1, shared.at[idxbuf.at[j]], add=True)

        return 0

    lax.fori_loop(0, STEPS, body, 0)
    plsc.subcore_barrier()

    for t in range(nchunk):
        pltpu.sync_copy(shared.at[pl.ds(sid * SUB + t * CHUNK, CHUNK)], rb0)
        pltpu.sync_copy(rb0,
                        acc_hbm.at[pl.ds(cid * NPAD + sid * SUB + t * CHUNK, CHUNK)])


def _sc_counts_body(dst_hbm, z128_hbm, ones_hbm, cnt_hbm,
                    idxbuf, rb0, onesbuf, shared, sem):
    cid = lax.axis_index("c")
    sid = lax.axis_index("s")
    wid = sid * 2 + cid
    nchunk = SUB // CHUNK

    pltpu.sync_copy(z128_hbm, rb0)
    for t in range(nchunk):
        pltpu.sync_copy(rb0, shared.at[pl.ds(sid * SUB + t * CHUNK, CHUNK)])
    pltpu.sync_copy(ones_hbm, onesbuf)
    pltpu.sync_copy(dst_hbm.at[wid], idxbuf)
    plsc.subcore_barrier()

    def body(j, _):
        pltpu.sync_copy(onesbuf, shared.at[idxbuf.at[j]], add=True)
        return 0

    lax.fori_loop(0, STEPS, body, 0)
    plsc.subcore_barrier()

    for t in range(nchunk):
        pltpu.sync_copy(shared.at[pl.ds(sid * SUB + t * CHUNK, CHUNK)], rb0)
        pltpu.sync_copy(rb0,
                        cnt_hbm.at[pl.ds(cid * NPAD + sid * SUB + t * CHUNK, CHUNK)])


def _sc_counts(dst3d):
    mesh = plsc.VectorSubcoreMesh(core_axis_name="c", subcore_axis_name="s")
    z128 = jnp.zeros((CHUNK, DIM), jnp.float32)
    ones = jnp.ones((CHUNK, DIM), jnp.float32)
    fn = functools.partial(
        pl.kernel, mesh=mesh,
        out_type=[
            jax.ShapeDtypeStruct((2 * NPAD, DIM), jnp.float32),
        ],
        scratch_types=[
            pltpu.VMEM((STEPS, CHUNK), jnp.int32),
            pltpu.VMEM((CHUNK, DIM), jnp.float32),
            pltpu.VMEM((CHUNK, DIM), jnp.float32),
            pltpu.VMEM_SHARED((NPAD, DIM), jnp.float32),
            pltpu.SemaphoreType.DMA,
        ],
    )(_sc_counts_body)
    (cacc,) = fn(dst3d, z128, ones)
    return cacc.reshape(2, NPAD, DIM)[:, :N, 0].sum(0)


def _sc_scatter(m, dst, dst3d):
    mesh = plsc.VectorSubcoreMesh(core_axis_name="c", subcore_axis_name="s")
    z128 = jnp.zeros((CHUNK, DIM), jnp.float32)
    fn = functools.partial(
        pl.kernel, mesh=mesh,
        out_type=[
            jax.ShapeDtypeStruct((2 * NPAD, DIM), jnp.float32),
        ],
        scratch_types=[
            pltpu.VMEM((STEPS, CHUNK), jnp.int32),
            pltpu.VMEM((CHUNK, DIM), jnp.float32),
            pltpu.VMEM((CHUNK, DIM), jnp.float32),
            pltpu.VMEM_SHARED((NPAD, DIM), jnp.float32),
            pltpu.SemaphoreType.DMA,
            pltpu.SemaphoreType.DMA,
        ],
    )(_sc_scatter_body)
    (acc,) = fn(m, dst3d, z128)
    seg = acc.reshape(2, NPAD, DIM)[:, :N].sum(0)
    counts = _sc_counts(dst3d)
    return seg, counts


def _pass1_body(srcf_ref, dstf_ref, ef_ref, w_ref, b_ref,
                alpha_ref, vt_ref, stats_ref):
    srcf = srcf_ref[...]
    dstf = dstf_ref[...]
    ef = ef_ref[...]

    def mm(a, w):
        return jax.lax.dot_general(a.astype(jnp.bfloat16),
                                   w.astype(jnp.bfloat16),
                                   (((1,), (0,)), ((), ())),
                                   preferred_element_type=jnp.float32)

    q = mm(dstf, w_ref[0]) + b_ref[0]
    hk = mm(srcf, w_ref[1]) + mm(dstf, w_ref[2]) + mm(ef, w_ref[3]) + b_ref[1]
    kt = mm(hk * jax.nn.sigmoid(hk), w_ref[7]) + b_ref[3]
    hv = mm(srcf, w_ref[4]) + mm(dstf, w_ref[5]) + mm(ef, w_ref[6]) + b_ref[2]
    vt = mm(hv * jax.nn.sigmoid(hv), w_ref[8]) + b_ref[4]

    alpha = q * kt * (1.0 / np.sqrt(D))
    alpha_ref[...] = alpha.astype(jnp.bfloat16)
    vt_ref[...] = vt.astype(jnp.bfloat16)

    s1 = jnp.sum(alpha, axis=0)
    s2 = jnp.sum(alpha * alpha, axis=0)
    part = jnp.concatenate([s1[None], s2[None], jnp.zeros((6, DIM), jnp.float32)], 0)

    @pl.when(pl.program_id(0) == 0)
    def _():
        stats_ref[...] = part

    @pl.when(pl.program_id(0) > 0)
    def _():
        stats_ref[...] += part


def _edge_pass1(srcf, dstf, ef, Wmats, Bvecs):
    grid = (E // BLK,)
    blk = lambda i: (i, 0)
    return pl.pallas_call(
        _pass1_body,
        grid=grid,
        in_specs=[
            pl.BlockSpec((BLK, DIM), blk),
            pl.BlockSpec((BLK, DIM), blk),
            pl.BlockSpec((BLK, DIM), blk),
            pl.BlockSpec((9, DIM, DIM), lambda i: (0, 0, 0)),
            pl.BlockSpec((8, DIM), lambda i: (0, 0)),
        ],
        out_specs=[
            pl.BlockSpec((BLK, DIM), blk),
            pl.BlockSpec((BLK, DIM), blk),
            pl.BlockSpec((8, DIM), lambda i: (0, 0)),
        ],
        out_shape=[
            jax.ShapeDtypeStruct((E, DIM), jnp.bfloat16),
            jax.ShapeDtypeStruct((E, DIM), jnp.bfloat16),
            jax.ShapeDtypeStruct((8, DIM), jnp.float32),
        ],
    )(srcf, dstf, ef, Wmats, Bvecs)


def _pass2_body(alpha_ref, vt_ref, aff_ref, m_ref, stats_ref):
    alpha = alpha_ref[...].astype(jnp.float32)
    vt = vt_ref[...].astype(jnp.float32)
    m = vt * jax.nn.sigmoid(alpha * aff_ref[0] + aff_ref[1])
    m_ref[...] = m
    s1 = jnp.sum(m, axis=0)
    s2 = jnp.sum(m * m, axis=0)
    part = jnp.concatenate([s1[None], s2[None], jnp.zeros((6, DIM), jnp.float32)], 0)

    @pl.when(pl.program_id(0) == 0)
    def _():
        stats_ref[...] = part

    @pl.when(pl.program_id(0) > 0)
    def _():
        stats_ref[...] += part


def _edge_pass2(alpha, vt, aff):
    grid = (E // BLK,)
    blk = lambda i: (i, 0)
    return pl.pallas_call(
        _pass2_body,
        grid=grid,
        in_specs=[
            pl.BlockSpec((BLK, DIM), blk),
            pl.BlockSpec((BLK, DIM), blk),
            pl.BlockSpec((8, DIM), lambda i: (0, 0)),
        ],
        out_specs=[
            pl.BlockSpec((BLK, DIM), blk),
            pl.BlockSpec((8, DIM), lambda i: (0, 0)),
        ],
        out_shape=[
            jax.ShapeDtypeStruct((E, DIM), jnp.float32),
            jax.ShapeDtypeStruct((8, DIM), jnp.float32),
        ],
    )(alpha, vt, aff)


def _pass3_body(ef_ref, m_ref, aff_ref, out_ref):
    x = ef_ref[...] + m_ref[...] * aff_ref[0] + aff_ref[1]
    out_ref[...] = jax.nn.softplus(x)


def _edge_pass3(ef, m, aff):
    grid = (E // BLK,)
    blk = lambda i: (i, 0)
    return pl.pallas_call(
        _pass3_body,
        grid=grid,
        in_specs=[
            pl.BlockSpec((BLK, DIM), blk),
            pl.BlockSpec((BLK, DIM), blk),
            pl.BlockSpec((8, DIM), lambda i: (0, 0)),
        ],
        out_specs=pl.BlockSpec((BLK, DIM), blk),
        out_shape=jax.ShapeDtypeStruct((E, DIM), jnp.float32),
    )(ef, m, aff)


def kernel(atom_feature, edge_feature, params, edge_index):
    src = edge_index[0]
    dst = edge_index[1]

    Wmats, Bvecs = _fold_weights(params)

    src3d = src.reshape(NW, STEPS, CHUNK)
    dst3d = dst.reshape(NW, STEPS, CHUNK)
    srcf, dstf = _sc_gather(atom_feature, src3d, dst3d)

    alpha, vt, astats = _edge_pass1(srcf, dstf, edge_feature, Wmats, Bvecs)

    cnt_a = float(E * H)
    s1 = astats[0].reshape(H, D).sum(0)
    s2 = astats[1].reshape(H, D).sum(0)
    mean_a = s1 / cnt_a
    var_a = jnp.maximum(s2 / cnt_a - mean_a * mean_a, 0.0)
    sa = params["bn_att_g"] / jnp.sqrt(var_a + EPS)
    ta = params["bn_att_b"] - mean_a * sa
    aff_a = jnp.concatenate([jnp.tile(sa, H)[None], jnp.tile(ta, H)[None],
                             jnp.zeros((6, DIM), jnp.float32)], 0)

    m, mstats = _edge_pass2(alpha, vt, aff_a)

    mean_m = mstats[0] / float(E)
    var_m = jnp.maximum(mstats[1] / float(E) - mean_m * mean_m, 0.0)
    sm = params["bn_msg_g"] / jnp.sqrt(var_m + EPS)
    tm = params["bn_msg_b"] - mean_m * sm
    aff_m = jnp.concatenate([sm[None], tm[None], jnp.zeros((6, DIM), jnp.float32)], 0)

    edge_out = _edge_pass3(edge_feature, m, aff_m)

    seg, counts = _sc_scatter(m, dst, dst3d)
    atom_out = atom_feature + seg * sm + counts[:, None] * tm

    return atom_out, edge_out


# speedup vs baseline: 1.1330x; 1.1330x over previous
import functools

import jax
import jax.numpy as jnp
import numpy as np
from jax import lax
from jax.experimental import pallas as pl
from jax.experimental.pallas import tpu as pltpu
from jax.experimental.pallas import tpu_sc as plsc

N = 10000
E = 320000
DIM = 128
H = 8
D = DIM // H
BLK = 4000
EPS = 1e-5

NW = 32
ROWS_PER_TILE = E // NW
CHUNK = 80
STEPS = ROWS_PER_TILE // CHUNK


def _sc_gather_body(a_hbm, src_hbm, dst_hbm, srcf_hbm, dstf_hbm,
                    idxbuf, rb0, rb1, sem0, sem1):
    wid = lax.axis_index("s") * 2 + lax.axis_index("c")
    for idx_hbm, out_hbm in ((src_hbm, srcf_hbm), (dst_hbm, dstf_hbm)):
        pltpu.sync_copy(idx_hbm.at[wid], idxbuf)
        pltpu.async_copy(a_hbm.at[idxbuf.at[0]], rb0, sem0)

        def body(j, _):
            even = (j % 2) == 0
            more = j + 1 < STEPS
            out_at = out_hbm.at[pl.ds(wid * ROWS_PER_TILE + j * CHUNK, CHUNK)]

            @pl.when(even)
            def _():
                pltpu.make_async_copy(a_hbm.at[idxbuf.at[j]], rb0, sem0).wait()

                @pl.when(more)
                def _():
                    pltpu.async_copy(a_hbm.at[idxbuf.at[j + 1]], rb1, sem1)

                pltpu.sync_copy(rb0, out_at)

            @pl.when(jnp.logical_not(even))
            def _():
                pltpu.make_async_copy(a_hbm.at[idxbuf.at[j]], rb1, sem1).wait()

                @pl.when(more)
                def _():
                    pltpu.async_copy(a_hbm.at[idxbuf.at[j + 1]], rb0, sem0)

                pltpu.sync_copy(rb1, out_at)

            return 0

        lax.fori_loop(0, STEPS, body, 0)


def _sc_gather(atom, src2d, dst2d):
    mesh = plsc.VectorSubcoreMesh(core_axis_name="c", subcore_axis_name="s")
    fn = functools.partial(
        pl.kernel, mesh=mesh,
        out_type=[
            jax.ShapeDtypeStruct((E, DIM), jnp.float32),
            jax.ShapeDtypeStruct((E, DIM), jnp.float32),
        ],
        scratch_types=[
            pltpu.VMEM((STEPS, CHUNK), jnp.int32),
            pltpu.VMEM((CHUNK, DIM), jnp.float32),
            pltpu.VMEM((CHUNK, DIM), jnp.float32),
            pltpu.SemaphoreType.DMA,
            pltpu.SemaphoreType.DMA,
        ],
    )(_sc_gather_body)
    return fn(atom, src2d, dst2d)


def _block_diag8(x):
    out = jnp.zeros((DIM, DIM), jnp.float32)
    for h in range(H):
        out = out.at[h * D:(h + 1) * D, h * D:(h + 1) * D].set(x)
    return out


def _fold_weights(params):
    Wq, bq = params["Wq"], params["bq"]
    Wk, bk = params["Wk"], params["bk"]
    Wv, bv = params["Wv"], params["bv"]
    We, be = params["We"], params["be"]

    def fold(W1, b1, Wx, bx):
        U1a = _block_diag8(W1[:, 0:D].T)
        U1b = _block_diag8(W1[:, D:2 * D].T)
        U1c = _block_diag8(W1[:, 2 * D:3 * D].T)
        Msrc = Wx.T @ U1a
        Mdst = Wx.T @ U1b
        Medge = We.T @ U1c
        bias = bx @ U1a + bx @ U1b + be @ U1c + jnp.tile(b1, H)
        return Msrc, Mdst, Medge, bias

    M1, M2, M3, b_hk = fold(params["ku_W1"], params["ku_b1"], Wk, bk)
    M4, M5, M6, b_hv = fold(params["mu_W1"], params["mu_b1"], Wv, bv)
    BD2k = _block_diag8(params["ku_W2"].T)
    BD2v = _block_diag8(params["mu_W2"].T)
    b_kt = jnp.tile(params["ku_b2"], H)
    b_vt = jnp.tile(params["mu_b2"], H)

    Wmats = jnp.stack([Wq.T, M1, M2, M3, M4, M5, M6, BD2k, BD2v])
    Bvecs = jnp.stack([bq, b_hk, b_hv, b_kt, b_vt,
                       jnp.zeros_like(bq), jnp.zeros_like(bq),
                       jnp.zeros_like(bq)])
    return Wmats, Bvecs


NPAD = 10240
SUB = NPAD // 16


def _sc_scatter_body(m_hbm, dst_hbm, z128_hbm,
                     acc_hbm,
                     idxbuf, rb0, rb1, shared, sem0, sem1):
    cid = lax.axis_index("c")
    sid = lax.axis_index("s")
    wid = sid * 2 + cid
    nchunk = SUB // CHUNK

    pltpu.sync_copy(z128_hbm, rb0)
    for t in range(nchunk):
        pltpu.sync_copy(rb0, shared.at[pl.ds(sid * SUB + t * CHUNK, CHUNK)])
    pltpu.sync_copy(dst_hbm.at[wid], idxbuf)
    plsc.subcore_barrier()

    base = wid * ROWS_PER_TILE
    pltpu.async_copy(m_hbm.at[pl.ds(base, CHUNK)], rb0, sem0)

    def body(j, _):
        even = (j % 2) == 0
        more = j + 1 < STEPS

        @pl.when(even)
        def _():
            pltpu.make_async_copy(
                m_hbm.at[pl.ds(base + j * CHUNK, CHUNK)], rb0, sem0).wait()

            @pl.when(more)
            def _():
                pltpu.async_copy(
                    m_hbm.at[pl.ds(base + (j + 1) * CHUNK, CHUNK)], rb1, sem1)

            pltpu.sync_copy(rb0, shared.at[idxbuf.at[j]], add=True)

        @pl.when(jnp.logical_not(even))
        def _():
            pltpu.make_async_copy(
                m_hbm.at[pl.ds(base + j * CHUNK, CHUNK)], rb1, sem1).wait()

            @pl.when(more)
            def _():
                pltpu.async_copy(
                    m_hbm.at[pl.ds(base + (j + 1) * CHUNK, CHUNK)], rb0, sem0)

            pltpu.sync_copy(rb1, shared.at[idxbuf.at[j]], add=True)

        return 0

    lax.fori_loop(0, STEPS, body, 0)
    plsc.subcore_barrier()

    for t in range(nchunk):
        pltpu.sync_copy(shared.at[pl.ds(sid * SUB + t * CHUNK, CHUNK)], rb0)
        pltpu.sync_copy(rb0,
                        acc_hbm.at[pl.ds(cid * NPAD + sid * SUB + t * CHUNK, CHUNK)])


def _sc_counts_body(dst_hbm, z128_hbm, ones_hbm, cnt_hbm,
                    idxbuf, rb0, onesbuf, shared, sem):
    cid = lax.axis_index("c")
    sid = lax.axis_index("s")
    wid = sid * 2 + cid
    nchunk = SUB // CHUNK

    pltpu.sync_copy(z128_hbm, rb0)
    for t in range(nchunk):
        pltpu.sync_copy(rb0, shared.at[pl.ds(sid * SUB + t * CHUNK, CHUNK)])
    pltpu.sync_copy(ones_hbm, onesbuf)
    pltpu.sync_copy(dst_hbm.at[wid], idxbuf)
    plsc.subcore_barrier()

    def body(j, _):
        pltpu.sync_copy(onesbuf, shared.at[idxbuf.at[j]], add=True)
        return 0

    lax.fori_loop(0, STEPS, body, 0)
    plsc.subcore_barrier()

    for t in range(nchunk):
        pltpu.sync_copy(shared.at[pl.ds(sid * SUB + t * CHUNK, CHUNK)], rb0)
        pltpu.sync_copy(rb0,
                        cnt_hbm.at[pl.ds(cid * NPAD + sid * SUB + t * CHUNK, CHUNK)])


def _sc_counts(dst3d):
    mesh = plsc.VectorSubcoreMesh(core_axis_name="c", subcore_axis_name="s")
    z128 = jnp.zeros((CHUNK, DIM), jnp.float32)
    ones = jnp.ones((CHUNK, DIM), jnp.float32)
    fn = functools.partial(
        pl.kernel, mesh=mesh,
        out_type=[
            jax.ShapeDtypeStruct((2 * NPAD, DIM), jnp.float32),
        ],
        scratch_types=[
            pltpu.VMEM((STEPS, CHUNK), jnp.int32),
            pltpu.VMEM((CHUNK, DIM), jnp.float32),
            pltpu.VMEM((CHUNK, DIM), jnp.float32),
            pltpu.VMEM_SHARED((NPAD, DIM), jnp.float32),
            pltpu.SemaphoreType.DMA,
        ],
    )(_sc_counts_body)
    (cacc,) = fn(dst3d, z128, ones)
    return cacc.reshape(2, NPAD, DIM)[:, :N, 0].sum(0)


def _sc_scatter(m, dst, dst3d):
    mesh = plsc.VectorSubcoreMesh(core_axis_name="c", subcore_axis_name="s")
    z128 = jnp.zeros((CHUNK, DIM), jnp.float32)
    fn = functools.partial(
        pl.kernel, mesh=mesh,
        out_type=[
            jax.ShapeDtypeStruct((2 * NPAD, DIM), jnp.float32),
        ],
        scratch_types=[
            pltpu.VMEM((STEPS, CHUNK), jnp.int32),
            pltpu.VMEM((CHUNK, DIM), jnp.float32),
            pltpu.VMEM((CHUNK, DIM), jnp.float32),
            pltpu.VMEM_SHARED((NPAD, DIM), jnp.float32),
            pltpu.SemaphoreType.DMA,
            pltpu.SemaphoreType.DMA,
        ],
    )(_sc_scatter_body)
    (acc,) = fn(m, dst3d, z128)
    seg = acc.reshape(2, NPAD, DIM)[:, :N].sum(0)
    counts = _sc_counts(dst3d)
    return seg, counts


def _pass1_body(srcf_ref, dstf_ref, ef_ref, w_ref, b_ref,
                alpha_ref, vt_ref, stats_ref):
    srcf = srcf_ref[...]
    dstf = dstf_ref[...]
    ef = ef_ref[...]

    def mm(a, w):
        return jax.lax.dot_general(a, w, (((1,), (0,)), ((), ())),
                                   preferred_element_type=jnp.float32)

    q = mm(dstf, w_ref[0]) + b_ref[0]
    hk = mm(srcf, w_ref[1]) + mm(dstf, w_ref[2]) + mm(ef, w_ref[3]) + b_ref[1]
    kt = mm(hk * jax.nn.sigmoid(hk), w_ref[7]) + b_ref[3]
    hv = mm(srcf, w_ref[4]) + mm(dstf, w_ref[5]) + mm(ef, w_ref[6]) + b_ref[2]
    vt = mm(hv * jax.nn.sigmoid(hv), w_ref[8]) + b_ref[4]

    alpha = q * kt * (1.0 / np.sqrt(D))
    alpha_ref[...] = alpha.astype(jnp.bfloat16)
    vt_ref[...] = vt.astype(jnp.bfloat16)

    s1 = jnp.sum(alpha, axis=0)
    s2 = jnp.sum(alpha * alpha, axis=0)
    part = jnp.concatenate([s1[None], s2[None], jnp.zeros((6, DIM), jnp.float32)], 0)

    @pl.when(pl.program_id(0) == 0)
    def _():
        stats_ref[...] = part

    @pl.when(pl.program_id(0) > 0)
    def _():
        stats_ref[...] += part


def _edge_pass1(srcf, dstf, ef, Wmats, Bvecs):
    grid = (E // BLK,)
    blk = lambda i: (i, 0)
    return pl.pallas_call(
        _pass1_body,
        grid=grid,
        in_specs=[
            pl.BlockSpec((BLK, DIM), blk),
            pl.BlockSpec((BLK, DIM), blk),
            pl.BlockSpec((BLK, DIM), blk),
            pl.BlockSpec((9, DIM, DIM), lambda i: (0, 0, 0)),
            pl.BlockSpec((8, DIM), lambda i: (0, 0)),
        ],
        out_specs=[
            pl.BlockSpec((BLK, DIM), blk),
            pl.BlockSpec((BLK, DIM), blk),
            pl.BlockSpec((8, DIM), lambda i: (0, 0)),
        ],
        out_shape=[
            jax.ShapeDtypeStruct((E, DIM), jnp.bfloat16),
            jax.ShapeDtypeStruct((E, DIM), jnp.bfloat16),
            jax.ShapeDtypeStruct((8, DIM), jnp.float32),
        ],
    )(srcf, dstf, ef, Wmats, Bvecs)


def _pass2_body(alpha_ref, vt_ref, aff_ref, m_ref, stats_ref):
    alpha = alpha_ref[...].astype(jnp.float32)
    vt = vt_ref[...].astype(jnp.float32)
    m = vt * jax.nn.sigmoid(alpha * aff_ref[0] + aff_ref[1])
    m_ref[...] = m
    s1 = jnp.sum(m, axis=0)
    s2 = jnp.sum(m * m, axis=0)
    part = jnp.concatenate([s1[None], s2[None], jnp.zeros((6, DIM), jnp.float32)], 0)

    @pl.when(pl.program_id(0) == 0)
    def _():
        stats_ref[...] = part

    @pl.when(pl.program_id(0) > 0)
    def _():
        stats_ref[...] += part


def _edge_pass2(alpha, vt, aff):
    grid = (E // BLK,)
    blk = lambda i: (i, 0)
    return pl.pallas_call(
        _pass2_body,
        grid=grid,
        in_specs=[
            pl.BlockSpec((BLK, DIM), blk),
            pl.BlockSpec((BLK, DIM), blk),
            pl.BlockSpec((8, DIM), lambda i: (0, 0)),
        ],
        out_specs=[
            pl.BlockSpec((BLK, DIM), blk),
            pl.BlockSpec((8, DIM), lambda i: (0, 0)),
        ],
        out_shape=[
            jax.ShapeDtypeStruct((E, DIM), jnp.float32),
            jax.ShapeDtypeStruct((8, DIM), jnp.float32),
        ],
    )(alpha, vt, aff)


def _pass3_body(ef_ref, m_ref, aff_ref, out_ref):
    x = ef_ref[...] + m_ref[...] * aff_ref[0] + aff_ref[1]
    out_ref[...] = jax.nn.softplus(x)


def _edge_pass3(ef, m, aff):
    grid = (E // BLK,)
    blk = lambda i: (i, 0)
    return pl.pallas_call(
        _pass3_body,
        grid=grid,
        in_specs=[
            pl.BlockSpec((BLK, DIM), blk),
            pl.BlockSpec((BLK, DIM), blk),
            pl.BlockSpec((8, DIM), lambda i: (0, 0)),
        ],
        out_specs=pl.BlockSpec((BLK, DIM), blk),
        out_shape=jax.ShapeDtypeStruct((E, DIM), jnp.float32),
    )(ef, m, aff)


def kernel(atom_feature, edge_feature, params, edge_index):
    src = edge_index[0]
    dst = edge_index[1]

    Wmats, Bvecs = _fold_weights(params)

    src3d = src.reshape(NW, STEPS, CHUNK)
    dst3d = dst.reshape(NW, STEPS, CHUNK)
    srcf, dstf = _sc_gather(atom_feature, src3d, dst3d)

    alpha, vt, astats = _edge_pass1(srcf, dstf, edge_feature, Wmats, Bvecs)

    cnt_a = float(E * H)
    s1 = astats[0].reshape(H, D).sum(0)
    s2 = astats[1].reshape(H, D).sum(0)
    mean_a = s1 / cnt_a
    var_a = jnp.maximum(s2 / cnt_a - mean_a * mean_a, 0.0)
    sa = params["bn_att_g"] / jnp.sqrt(var_a + EPS)
    ta = params["bn_att_b"] - mean_a * sa
    aff_a = jnp.concatenate([jnp.tile(sa, H)[None], jnp.tile(ta, H)[None],
                             jnp.zeros((6, DIM), jnp.float32)], 0)

    m, mstats = _edge_pass2(alpha, vt, aff_a)

    mean_m = mstats[0] / float(E)
    var_m = jnp.maximum(mstats[1] / float(E) - mean_m * mean_m, 0.0)
    sm = params["bn_msg_g"] / jnp.sqrt(var_m + EPS)
    tm = params["bn_msg_b"] - mean_m * sm
    aff_m = jnp.concatenate([sm[None], tm[None], jnp.zeros((6, DIM), jnp.float32)], 0)

    edge_out = _edge_pass3(edge_feature, m, aff_m)

    seg, counts = _sc_scatter(m, dst, dst3d)
    atom_out = atom_feature + seg * sm + counts[:, None] * tm

    return atom_out, edge_out


# speedup vs baseline: 1.1589x; 1.0228x over previous
import functools

import jax
import jax.numpy as jnp
import numpy as np
from jax import lax
from jax.experimental import pallas as pl
from jax.experimental.pallas import tpu as pltpu
from jax.experimental.pallas import tpu_sc as plsc

N = 10000
E = 320000
DIM = 128
H = 8
D = DIM // H
BLK = 8000
EPS = 1e-5

NW = 32
ROWS_PER_TILE = E // NW
CHUNK = 80
STEPS = ROWS_PER_TILE // CHUNK


def _sc_gather_body(a_hbm, src_hbm, dst_hbm, srcf_hbm, dstf_hbm,
                    idxbuf, rb0, rb1, sem0, sem1):
    wid = lax.axis_index("s") * 2 + lax.axis_index("c")
    for idx_hbm, out_hbm in ((src_hbm, srcf_hbm), (dst_hbm, dstf_hbm)):
        pltpu.sync_copy(idx_hbm.at[wid], idxbuf)
        pltpu.async_copy(a_hbm.at[idxbuf.at[0]], rb0, sem0)

        def body(j, _):
            even = (j % 2) == 0
            more = j + 1 < STEPS
            out_at = out_hbm.at[pl.ds(wid * ROWS_PER_TILE + j * CHUNK, CHUNK)]

            @pl.when(even)
            def _():
                pltpu.make_async_copy(a_hbm.at[idxbuf.at[j]], rb0, sem0).wait()

                @pl.when(more)
                def _():
                    pltpu.async_copy(a_hbm.at[idxbuf.at[j + 1]], rb1, sem1)

                pltpu.sync_copy(rb0, out_at)

            @pl.when(jnp.logical_not(even))
            def _():
                pltpu.make_async_copy(a_hbm.at[idxbuf.at[j]], rb1, sem1).wait()

                @pl.when(more)
                def _():
                    pltpu.async_copy(a_hbm.at[idxbuf.at[j + 1]], rb0, sem0)

                pltpu.sync_copy(rb1, out_at)

            return 0

        lax.fori_loop(0, STEPS, body, 0)


def _sc_gather(atom, src2d, dst2d):
    mesh = plsc.VectorSubcoreMesh(core_axis_name="c", subcore_axis_name="s")
    fn = functools.partial(
        pl.kernel, mesh=mesh,
        out_type=[
            jax.ShapeDtypeStruct((E, DIM), jnp.float32),
            jax.ShapeDtypeStruct((E, DIM), jnp.float32),
        ],
        scratch_types=[
            pltpu.VMEM((STEPS, CHUNK), jnp.int32),
            pltpu.VMEM((CHUNK, DIM), jnp.float32),
            pltpu.VMEM((CHUNK, DIM), jnp.float32),
            pltpu.SemaphoreType.DMA,
            pltpu.SemaphoreType.DMA,
        ],
    )(_sc_gather_body)
    return fn(atom, src2d, dst2d)


def _block_diag8(x):
    out = jnp.zeros((DIM, DIM), jnp.float32)
    for h in range(H):
        out = out.at[h * D:(h + 1) * D, h * D:(h + 1) * D].set(x)
    return out


def _fold_weights(params):
    Wq, bq = params["Wq"], params["bq"]
    Wk, bk = params["Wk"], params["bk"]
    Wv, bv = params["Wv"], params["bv"]
    We, be = params["We"], params["be"]

    def fold(W1, b1, Wx, bx):
        U1a = _block_diag8(W1[:, 0:D].T)
        U1b = _block_diag8(W1[:, D:2 * D].T)
        U1c = _block_diag8(W1[:, 2 * D:3 * D].T)
        Msrc = Wx.T @ U1a
        Mdst = Wx.T @ U1b
        Medge = We.T @ U1c
        bias = bx @ U1a + bx @ U1b + be @ U1c + jnp.tile(b1, H)
        return Msrc, Mdst, Medge, bias

    M1, M2, M3, b_hk = fold(params["ku_W1"], params["ku_b1"], Wk, bk)
    M4, M5, M6, b_hv = fold(params["mu_W1"], params["mu_b1"], Wv, bv)
    BD2k = _block_diag8(params["ku_W2"].T)
    BD2v = _block_diag8(params["mu_W2"].T)
    b_kt = jnp.tile(params["ku_b2"], H)
    b_vt = jnp.tile(params["mu_b2"], H)

    Wmats = jnp.stack([Wq.T, M1, M2, M3, M4, M5, M6, BD2k, BD2v])
    Bvecs = jnp.stack([bq, b_hk, b_hv, b_kt, b_vt,
                       jnp.zeros_like(bq), jnp.zeros_like(bq),
                       jnp.zeros_like(bq)])
    return Wmats, Bvecs


NPAD = 10240
SUB = NPAD // 16


def _sc_scatter_body(m_hbm, dst_hbm, z128_hbm,
                     acc_hbm,
                     idxbuf, rb0, rb1, shared, sem0, sem1):
    cid = lax.axis_index("c")
    sid = lax.axis_index("s")
    wid = sid * 2 + cid
    nchunk = SUB // CHUNK

    pltpu.sync_copy(z128_hbm, rb0)
    for t in range(nchunk):
        pltpu.sync_copy(rb0, shared.at[pl.ds(sid * SUB + t * CHUNK, CHUNK)])
    pltpu.sync_copy(dst_hbm.at[wid], idxbuf)
    plsc.subcore_barrier()

    base = wid * ROWS_PER_TILE
    pltpu.async_copy(m_hbm.at[pl.ds(base, CHUNK)], rb0, sem0)

    def body(j, _):
        even = (j % 2) == 0
        more = j + 1 < STEPS

        @pl.when(even)
        def _():
            pltpu.make_async_copy(
                m_hbm.at[pl.ds(base + j * CHUNK, CHUNK)], rb0, sem0).wait()

            @pl.when(more)
            def _():
                pltpu.async_copy(
                    m_hbm.at[pl.ds(base + (j + 1) * CHUNK, CHUNK)], rb1, sem1)

            pltpu.sync_copy(rb0, shared.at[idxbuf.at[j]], add=True)

        @pl.when(jnp.logical_not(even))
        def _():
            pltpu.make_async_copy(
                m_hbm.at[pl.ds(base + j * CHUNK, CHUNK)], rb1, sem1).wait()

            @pl.when(more)
            def _():
                pltpu.async_copy(
                    m_hbm.at[pl.ds(base + (j + 1) * CHUNK, CHUNK)], rb0, sem0)

            pltpu.sync_copy(rb1, shared.at[idxbuf.at[j]], add=True)

        return 0

    lax.fori_loop(0, STEPS, body, 0)
    plsc.subcore_barrier()

    for t in range(nchunk):
        pltpu.sync_copy(shared.at[pl.ds(sid * SUB + t * CHUNK, CHUNK)], rb0)
        pltpu.sync_copy(rb0,
                        acc_hbm.at[pl.ds(cid * NPAD + sid * SUB + t * CHUNK, CHUNK)])


def _sc_counts_body(dst_hbm, z128_hbm, ones_hbm, cnt_hbm,
                    idxbuf, rb0, onesbuf, shared, sem):
    cid = lax.axis_index("c")
    sid = lax.axis_index("s")
    wid = sid * 2 + cid
    nchunk = SUB // CHUNK

    pltpu.sync_copy(z128_hbm, rb0)
    for t in range(nchunk):
        pltpu.sync_copy(rb0, shared.at[pl.ds(sid * SUB + t * CHUNK, CHUNK)])
    pltpu.sync_copy(ones_hbm, onesbuf)
    pltpu.sync_copy(dst_hbm.at[wid], idxbuf)
    plsc.subcore_barrier()

    def body(j, _):
        pltpu.sync_copy(onesbuf, shared.at[idxbuf.at[j]], add=True)
        return 0

    lax.fori_loop(0, STEPS, body, 0)
    plsc.subcore_barrier()

    for t in range(nchunk):
        pltpu.sync_copy(shared.at[pl.ds(sid * SUB + t * CHUNK, CHUNK)], rb0)
        pltpu.sync_copy(rb0,
                        cnt_hbm.at[pl.ds(cid * NPAD + sid * SUB + t * CHUNK, CHUNK)])


def _sc_counts(dst3d):
    mesh = plsc.VectorSubcoreMesh(core_axis_name="c", subcore_axis_name="s")
    z128 = jnp.zeros((CHUNK, DIM), jnp.float32)
    ones = jnp.ones((CHUNK, DIM), jnp.float32)
    fn = functools.partial(
        pl.kernel, mesh=mesh,
        out_type=[
            jax.ShapeDtypeStruct((2 * NPAD, DIM), jnp.float32),
        ],
        scratch_types=[
            pltpu.VMEM((STEPS, CHUNK), jnp.int32),
            pltpu.VMEM((CHUNK, DIM), jnp.float32),
            pltpu.VMEM((CHUNK, DIM), jnp.float32),
            pltpu.VMEM_SHARED((NPAD, DIM), jnp.float32),
            pltpu.SemaphoreType.DMA,
        ],
    )(_sc_counts_body)
    (cacc,) = fn(dst3d, z128, ones)
    return cacc.reshape(2, NPAD, DIM)[:, :N, 0].sum(0)


def _sc_scatter(m, dst, dst3d):
    mesh = plsc.VectorSubcoreMesh(core_axis_name="c", subcore_axis_name="s")
    z128 = jnp.zeros((CHUNK, DIM), jnp.float32)
    fn = functools.partial(
        pl.kernel, mesh=mesh,
        out_type=[
            jax.ShapeDtypeStruct((2 * NPAD, DIM), jnp.float32),
        ],
        scratch_types=[
            pltpu.VMEM((STEPS, CHUNK), jnp.int32),
            pltpu.VMEM((CHUNK, DIM), jnp.float32),
            pltpu.VMEM((CHUNK, DIM), jnp.float32),
            pltpu.VMEM_SHARED((NPAD, DIM), jnp.float32),
            pltpu.SemaphoreType.DMA,
            pltpu.SemaphoreType.DMA,
        ],
    )(_sc_scatter_body)
    (acc,) = fn(m, dst3d, z128)
    seg = acc.reshape(2, NPAD, DIM)[:, :N].sum(0)
    counts = _sc_counts(dst3d)
    return seg, counts


def _pass1_body(srcf_ref, dstf_ref, ef_ref, w_ref, b_ref,
                alpha_ref, vt_ref, stats_ref):
    srcf = srcf_ref[...]
    dstf = dstf_ref[...]
    ef = ef_ref[...]

    def mm(a, w):
        return jax.lax.dot_general(a, w, (((1,), (0,)), ((), ())),
                                   preferred_element_type=jnp.float32)

    q = mm(dstf, w_ref[0]) + b_ref[0]
    hk = mm(srcf, w_ref[1]) + mm(dstf, w_ref[2]) + mm(ef, w_ref[3]) + b_ref[1]
    kt = mm(hk * jax.nn.sigmoid(hk), w_ref[7]) + b_ref[3]
    hv = mm(srcf, w_ref[4]) + mm(dstf, w_ref[5]) + mm(ef, w_ref[6]) + b_ref[2]
    vt = mm(hv * jax.nn.sigmoid(hv), w_ref[8]) + b_ref[4]

    alpha = q * kt * (1.0 / np.sqrt(D))
    alpha_ref[...] = alpha.astype(jnp.bfloat16)
    vt_ref[...] = vt.astype(jnp.bfloat16)

    s1 = jnp.sum(alpha, axis=0)
    s2 = jnp.sum(alpha * alpha, axis=0)
    part = jnp.concatenate([s1[None], s2[None], jnp.zeros((6, DIM), jnp.float32)], 0)

    @pl.when(pl.program_id(0) == 0)
    def _():
        stats_ref[...] = part

    @pl.when(pl.program_id(0) > 0)
    def _():
        stats_ref[...] += part


def _edge_pass1(srcf, dstf, ef, Wmats, Bvecs):
    grid = (E // BLK,)
    blk = lambda i: (i, 0)
    return pl.pallas_call(
        _pass1_body,
        grid=grid,
        in_specs=[
            pl.BlockSpec((BLK, DIM), blk),
            pl.BlockSpec((BLK, DIM), blk),
            pl.BlockSpec((BLK, DIM), blk),
            pl.BlockSpec((9, DIM, DIM), lambda i: (0, 0, 0)),
            pl.BlockSpec((8, DIM), lambda i: (0, 0)),
        ],
        out_specs=[
            pl.BlockSpec((BLK, DIM), blk),
            pl.BlockSpec((BLK, DIM), blk),
            pl.BlockSpec((8, DIM), lambda i: (0, 0)),
        ],
        out_shape=[
            jax.ShapeDtypeStruct((E, DIM), jnp.bfloat16),
            jax.ShapeDtypeStruct((E, DIM), jnp.bfloat16),
            jax.ShapeDtypeStruct((8, DIM), jnp.float32),
        ],
    )(srcf, dstf, ef, Wmats, Bvecs)


def _pass2_body(alpha_ref, vt_ref, aff_ref, m_ref, stats_ref):
    alpha = alpha_ref[...].astype(jnp.float32)
    vt = vt_ref[...].astype(jnp.float32)
    m = vt * jax.nn.sigmoid(alpha * aff_ref[0] + aff_ref[1])
    m_ref[...] = m
    s1 = jnp.sum(m, axis=0)
    s2 = jnp.sum(m * m, axis=0)
    part = jnp.concatenate([s1[None], s2[None], jnp.zeros((6, DIM), jnp.float32)], 0)

    @pl.when(pl.program_id(0) == 0)
    def _():
        stats_ref[...] = part

    @pl.when(pl.program_id(0) > 0)
    def _():
        stats_ref[...] += part


def _edge_pass2(alpha, vt, aff):
    grid = (E // BLK,)
    blk = lambda i: (i, 0)
    return pl.pallas_call(
        _pass2_body,
        grid=grid,
        in_specs=[
            pl.BlockSpec((BLK, DIM), blk),
            pl.BlockSpec((BLK, DIM), blk),
            pl.BlockSpec((8, DIM), lambda i: (0, 0)),
        ],
        out_specs=[
            pl.BlockSpec((BLK, DIM), blk),
            pl.BlockSpec((8, DIM), lambda i: (0, 0)),
        ],
        out_shape=[
            jax.ShapeDtypeStruct((E, DIM), jnp.float32),
            jax.ShapeDtypeStruct((8, DIM), jnp.float32),
        ],
    )(alpha, vt, aff)


def _pass3_body(ef_ref, m_ref, aff_ref, out_ref):
    x = ef_ref[...] + m_ref[...] * aff_ref[0] + aff_ref[1]
    out_ref[...] = jax.nn.softplus(x)


def _edge_pass3(ef, m, aff):
    grid = (E // BLK,)
    blk = lambda i: (i, 0)
    return pl.pallas_call(
        _pass3_body,
        grid=grid,
        in_specs=[
            pl.BlockSpec((BLK, DIM), blk),
            pl.BlockSpec((BLK, DIM), blk),
            pl.BlockSpec((8, DIM), lambda i: (0, 0)),
        ],
        out_specs=pl.BlockSpec((BLK, DIM), blk),
        out_shape=jax.ShapeDtypeStruct((E, DIM), jnp.float32),
    )(ef, m, aff)


def kernel(atom_feature, edge_feature, params, edge_index):
    src = edge_index[0]
    dst = edge_index[1]

    Wmats, Bvecs = _fold_weights(params)

    src3d = src.reshape(NW, STEPS, CHUNK)
    dst3d = dst.reshape(NW, STEPS, CHUNK)
    srcf, dstf = _sc_gather(atom_feature, src3d, dst3d)

    alpha, vt, astats = _edge_pass1(srcf, dstf, edge_feature, Wmats, Bvecs)

    cnt_a = float(E * H)
    s1 = astats[0].reshape(H, D).sum(0)
    s2 = astats[1].reshape(H, D).sum(0)
    mean_a = s1 / cnt_a
    var_a = jnp.maximum(s2 / cnt_a - mean_a * mean_a, 0.0)
    sa = params["bn_att_g"] / jnp.sqrt(var_a + EPS)
    ta = params["bn_att_b"] - mean_a * sa
    aff_a = jnp.concatenate([jnp.tile(sa, H)[None], jnp.tile(ta, H)[None],
                             jnp.zeros((6, DIM), jnp.float32)], 0)

    m, mstats = _edge_pass2(alpha, vt, aff_a)

    mean_m = mstats[0] / float(E)
    var_m = jnp.maximum(mstats[1] / float(E) - mean_m * mean_m, 0.0)
    sm = params["bn_msg_g"] / jnp.sqrt(var_m + EPS)
    tm = params["bn_msg_b"] - mean_m * sm
    aff_m = jnp.concatenate([sm[None], tm[None], jnp.zeros((6, DIM), jnp.float32)], 0)

    edge_out = _edge_pass3(edge_feature, m, aff_m)

    seg, counts = _sc_scatter(m, dst, dst3d)
    atom_out = atom_feature + seg * sm + counts[:, None] * tm

    return atom_out, edge_out
